# baseline (speedup 1.0000x reference)
import functools

import jax
import jax.numpy as jnp
from jax import lax
from jax.experimental import pallas as pl
from jax.experimental.pallas import tpu as pltpu
from jax.experimental.pallas import tpu_sc as plsc

N_NODES = 10000
N_PAD = 10240
N_EDGES = 320000
D_IN = 128
D_HID = 256
D_HALF = 128
BN_EPS = 1e-5

NC = 2
NS = 16
K = 80
KA = 128
CHUNKS = 160
E_TILE_PAD = CHUNKS * KA
NBUF = 4

E_PER_TILE_DEG = N_EDGES // (NC * NS)
E_PER_TILE_AGG = N_EDGES // NS
ZROWS = N_PAD // NS


def _fill_zero_2d(buf, rows, cols):
    z = jnp.zeros((16,), jnp.float32)

    @pl.loop(0, rows)
    def _(i):
        for j in range(cols // 16):
            buf[i, pl.ds(16 * j, 16)] = z


def _sc_mesh():
    return plsc.VectorSubcoreMesh(core_axis_name="c", subcore_axis_name="s")


@functools.partial(
    pl.kernel,
    out_type=jax.ShapeDtypeStruct((NC * N_PAD,), jnp.float32),
    mesh=_sc_mesh(),
    scratch_types=[
        pltpu.VMEM((K,), jnp.int32),
        pltpu.VMEM((K,), jnp.float32),
        pltpu.VMEM((ZROWS,), jnp.float32),
        pltpu.VMEM_SHARED((N_PAD,), jnp.float32),
    ],
)
def _sc_degree(dst_hbm, out_hbm, dstv, onesv, zbuf, acc):
    c = lax.axis_index("c")
    s = lax.axis_index("s")

    one = jnp.ones((16,), jnp.float32)
    zero = jnp.zeros((16,), jnp.float32)
    for j in range(K // 16):
        onesv[pl.ds(16 * j, 16)] = one

    @pl.loop(0, ZROWS // 16)
    def _(i):
        zbuf[pl.ds(16 * i, 16)] = zero

    pltpu.sync_copy(zbuf, acc.at[pl.ds(s * ZROWS, ZROWS)])
    plsc.subcore_barrier()

    base0 = (c * NS + s) * E_PER_TILE_DEG

    @pl.loop(0, E_PER_TILE_DEG // K)
    def _(j):
        pltpu.sync_copy(dst_hbm.at[pl.ds(base0 + j * K, K)], dstv)
        pltpu.sync_copy(onesv, acc.at[dstv], add=True)

    plsc.subcore_barrier()
    pltpu.sync_copy(acc.at[pl.ds(s * ZROWS, ZROWS)],
                    out_hbm.at[pl.ds(c * N_PAD + s * ZROWS, ZROWS)])


NIB = 4

_AGG_SCRATCH = (
    [pltpu.VMEM((2, KA), jnp.int32) for _ in range(NIB)]
    + [pltpu.VMEM((KA, D_HALF), jnp.float32) for _ in range(2)]
    + [pltpu.VMEM_SHARED((N_PAD, D_HALF), jnp.float32)]
    + [pltpu.SemaphoreType.DMA for _ in range(NIB + 6)]
)


@functools.partial(
    pl.kernel,
    out_type=jax.ShapeDtypeStruct((NC, N_PAD, D_HALF), jnp.float32),
    mesh=_sc_mesh(),
    scratch_types=_AGG_SCRATCH,
)
def _sc_aggregate(hp_hbm, comb_hbm, out_hbm, *sc):
    ib = sc[0:NIB]
    rows = sc[NIB:NIB + 2]
    acc = sc[NIB + 2]
    isem = sc[NIB + 3:2 * NIB + 3]
    gsem = sc[2 * NIB + 3:2 * NIB + 7]
    ssem = sc[2 * NIB + 7:2 * NIB + 9]

    c = lax.axis_index("c")
    s = lax.axis_index("s")

    _fill_zero_2d(rows[0], KA, D_HALF)
    for r in range(ZROWS // KA):
        pltpu.sync_copy(rows[0], acc.at[pl.ds(s * ZROWS + r * KA, KA)])
    plsc.subcore_barrier()

    tbase = (c * NS + s) * CHUNKS

    def fire_idx(j, q):
        pltpu.async_copy(comb_hbm.at[tbase + j], ib[q], isem[q])

    def wait_idx(q):
        pltpu.make_async_copy(comb_hbm.at[tbase], ib[q], isem[q]).wait()

    H = KA // 2

    def fire_gather(b, q):
        pltpu.async_copy(hp_hbm.at[ib[q].at[0, pl.ds(0, H)]],
                         rows[b].at[pl.ds(0, H)], gsem[2 * b])
        pltpu.async_copy(hp_hbm.at[ib[q].at[0, pl.ds(H, H)]],
                         rows[b].at[pl.ds(H, H)], gsem[2 * b + 1])

    def wait_gather(b, q):
        pltpu.make_async_copy(hp_hbm.at[ib[q].at[0, pl.ds(0, H)]],
                              rows[b].at[pl.ds(0, H)], gsem[2 * b]).wait()
        pltpu.make_async_copy(hp_hbm.at[ib[q].at[0, pl.ds(H, H)]],
                              rows[b].at[pl.ds(H, H)], gsem[2 * b + 1]).wait()

    def fire_scatter(b, q):
        pltpu.async_copy(rows[b], acc.at[ib[q].at[1]], ssem[b], add=True)

    def wait_scatter(b, q):
        pltpu.make_async_copy(rows[b], acc.at[ib[q].at[1]], ssem[b]).wait()

    def step(j, u, do_idx=True, do_gather=True, do_wait_prev=True):
        b, q = u % 2, u % 4
        bn, qn, qf = (u + 1) % 2, (u + 1) % 4, (u + 2) % 4
        wait_gather(b, q)
        fire_scatter(b, q)
        if do_idx:
            fire_idx(j + 2, qf)
        if do_wait_prev:
            wait_scatter(bn, qn)
        if do_gather:
            wait_idx(qn)
            fire_gather(bn, qn)

    fire_idx(0, 0)
    fire_idx(1, 1)
    wait_idx(0)
    fire_gather(0, 0)

    step(0, 0, do_wait_prev=False)
    for u in range(1, 4):
        step(u, u)

    @pl.loop(4, CHUNKS - 4, step=4)
    def _(j0):
        for u in range(4):
            step(j0 + u, u)

    step(CHUNKS - 4, 0)
    step(CHUNKS - 3, 1)
    step(CHUNKS - 2, 2, do_idx=False)
    step(CHUNKS - 1, 3, do_idx=False, do_gather=False)
    wait_scatter(1, 3)

    plsc.subcore_barrier()
    pltpu.sync_copy(acc.at[pl.ds(s * ZROWS, ZROWS)],
                    out_hbm.at[c, pl.ds(s * ZROWS, ZROWS)])


def _dinv_from(degp_ref):
    deg = degp_ref[:, 0:1] + degp_ref[:, 1:2] + 1.0
    return lax.rsqrt(deg)


def _tc1_body(x_ref, w1_ref, degp_ref, out_ref):
    dinv = _dinv_from(degp_ref)
    h = jnp.dot(x_ref[...], w1_ref[...], preferred_element_type=jnp.float32)
    hp = h * dinv
    out_ref[0] = hp[:, :D_HALF]
    out_ref[1] = hp[:, D_HALF:]


def _bn_relu(a, g_ref, be_ref):
    mean = jnp.mean(a, axis=0, keepdims=True)
    var = jnp.mean((a - mean) * (a - mean), axis=0, keepdims=True)
    zn = (a - mean) * lax.rsqrt(var + BN_EPS)
    return jnp.maximum(zn * g_ref[...][None, :] + be_ref[...][None, :], 0.0)


def _pre_bn(agg_ref, hp_ref, dinv, b_ref):
    a_lo = (agg_ref[0, :N_NODES, :] + hp_ref[0]) * dinv
    a_hi = (agg_ref[1, :N_NODES, :] + hp_ref[1]) * dinv
    return jnp.concatenate([a_lo, a_hi], axis=1) + b_ref[...][None, :]


def _tc2_body(agg_ref, hp_ref, degp_ref, b1_ref, g1_ref, be1_ref, w2_ref,
              out_ref):
    dinv = _dinv_from(degp_ref)
    a = _pre_bn(agg_ref, hp_ref, dinv, b1_ref)
    z = _bn_relu(a, g1_ref, be1_ref)
    h2 = jnp.dot(z, w2_ref[...], preferred_element_type=jnp.float32)
    hp2 = h2 * dinv
    out_ref[0] = hp2[:, :D_HALF]
    out_ref[1] = hp2[:, D_HALF:]


def _tc3_body(agg_ref, hp_ref, degp_ref, b2_ref, g2_ref, be2_ref, out_ref):
    dinv = _dinv_from(degp_ref)
    a = _pre_bn(agg_ref, hp_ref, dinv, b2_ref)
    out_ref[...] = _bn_relu(a, g2_ref, be2_ref)


def _tc_call(body, n_in, out_shape):
    return pl.pallas_call(
        body,
        out_shape=out_shape,
        in_specs=[pl.BlockSpec(memory_space=pltpu.VMEM)] * n_in,
        out_specs=pl.BlockSpec(memory_space=pltpu.VMEM)
        if not isinstance(out_shape, (list, tuple)) else
        [pl.BlockSpec(memory_space=pltpu.VMEM)] * len(out_shape),
    )


def kernel(x, edge_index, W1, b1, g1, be1, W2, b2, g2, be2):
    src = edge_index[0].astype(jnp.int32)
    dst = edge_index[1].astype(jnp.int32)

    pad = E_TILE_PAD - E_PER_TILE_AGG
    srcp = jnp.pad(src.reshape(NS, E_PER_TILE_AGG), ((0, 0), (0, pad)),
                   constant_values=0).reshape(NS, CHUNKS, KA)
    dstp = jnp.pad(dst.reshape(NS, E_PER_TILE_AGG), ((0, 0), (0, pad)),
                   constant_values=N_NODES).reshape(NS, CHUNKS, KA)
    comb = jnp.stack([
        jnp.stack([srcp, dstp], axis=2),
        jnp.stack([srcp + N_NODES, dstp], axis=2),
    ]).reshape(NC * NS * CHUNKS, 2, KA)

    degp = _sc_degree(dst).reshape(NC, N_PAD)
    degp2 = degp[:, :N_NODES].T

    hp1 = _tc_call(_tc1_body, 3,
                   jax.ShapeDtypeStruct((NC, N_NODES, D_HALF), jnp.float32))(
                       x, W1, degp2)
    agg1 = _sc_aggregate(hp1.reshape(NC * N_NODES, D_HALF), comb)

    hp2 = _tc_call(_tc2_body, 7,
                   jax.ShapeDtypeStruct((NC, N_NODES, D_HALF), jnp.float32))(
                       agg1, hp1, degp2, b1, g1, be1, W2)
    agg2 = _sc_aggregate(hp2.reshape(NC * N_NODES, D_HALF), comb)

    out = _tc_call(_tc3_body, 6,
                   jax.ShapeDtypeStruct((N_NODES, D_HID), jnp.float32))(
                       agg2, hp2, degp2, b2, g2, be2)
    return out


# speedup vs baseline: 1.0316x; 1.0316x over previous
import functools

import jax
import jax.numpy as jnp
from jax import lax
from jax.experimental import pallas as pl
from jax.experimental.pallas import tpu as pltpu
from jax.experimental.pallas import tpu_sc as plsc

N_NODES = 10000
N_PAD = 10240
N_EDGES = 320000
D_IN = 128
D_HID = 256
D_HALF = 128
BN_EPS = 1e-5

NC = 2
NS = 16
K = 80
KA = 128
CHUNKS = 160
E_TILE_PAD = CHUNKS * KA
NBUF = 4

E_PER_TILE_DEG = N_EDGES // (NC * NS)
E_PER_TILE_AGG = N_EDGES // NS
ZROWS = N_PAD // NS


def _fill_zero_2d(buf, rows, cols):
    z = jnp.zeros((16,), jnp.float32)

    @pl.loop(0, rows)
    def _(i):
        for j in range(cols // 16):
            buf[i, pl.ds(16 * j, 16)] = z


def _sc_mesh():
    return plsc.VectorSubcoreMesh(core_axis_name="c", subcore_axis_name="s")


_DEG_CHUNKS = E_PER_TILE_DEG // K


@functools.partial(
    pl.kernel,
    out_type=jax.ShapeDtypeStruct((NC * N_PAD,), jnp.float32),
    mesh=_sc_mesh(),
    scratch_types=(
        [pltpu.VMEM((K,), jnp.int32) for _ in range(4)]
        + [pltpu.VMEM((K,), jnp.float32),
           pltpu.VMEM((ZROWS,), jnp.float32)]
        + [pltpu.VMEM_SHARED((N_PAD,), jnp.float32)]
        + [pltpu.SemaphoreType.DMA for _ in range(4)]
    ),
)
def _sc_degree(dst_hbm, out_hbm, *sc):
    dstv = sc[0:4]
    onesv, zbuf, acc = sc[4], sc[5], sc[6]
    isem = sc[7:11]

    c = lax.axis_index("c")
    s = lax.axis_index("s")

    one = jnp.ones((16,), jnp.float32)
    zero = jnp.zeros((16,), jnp.float32)
    for j in range(K // 16):
        onesv[pl.ds(16 * j, 16)] = one

    @pl.loop(0, ZROWS // 16)
    def _(i):
        zbuf[pl.ds(16 * i, 16)] = zero

    pltpu.sync_copy(zbuf, acc.at[pl.ds(s * ZROWS, ZROWS)])
    plsc.subcore_barrier()

    base0 = (c * NS + s) * E_PER_TILE_DEG

    def fire_idx(j, q):
        pltpu.async_copy(dst_hbm.at[pl.ds(base0 + j * K, K)], dstv[q],
                         isem[q])

    def step(j, q, do_idx=True):
        pltpu.make_async_copy(dst_hbm.at[pl.ds(base0, K)], dstv[q],
                              isem[q]).wait()
        pltpu.sync_copy(onesv, acc.at[dstv[q]], add=True)
        if do_idx:
            fire_idx(j + 3, q3(q))

    def q3(q):
        return (q + 3) % 4

    for q in range(3):
        fire_idx(q, q)
    step(0, 0)

    @pl.loop(1, _DEG_CHUNKS - 4, step=4)
    def _(j0):
        for u in range(4):
            step(j0 + u, (1 + u) % 4)

    for j in range(_DEG_CHUNKS - 4, _DEG_CHUNKS):
        step(j, j % 4, do_idx=(j + 3 < _DEG_CHUNKS))

    plsc.subcore_barrier()
    pltpu.sync_copy(acc.at[pl.ds(s * ZROWS, ZROWS)],
                    out_hbm.at[pl.ds(c * N_PAD + s * ZROWS, ZROWS)])


NIB = 4

_AGG_SCRATCH = (
    [pltpu.VMEM((2, KA), jnp.int32) for _ in range(NIB)]
    + [pltpu.VMEM((KA, D_HALF), jnp.float32) for _ in range(2)]
    + [pltpu.VMEM_SHARED((N_PAD, D_HALF), jnp.float32)]
    + [pltpu.SemaphoreType.DMA for _ in range(NIB + 4)]
)


@functools.partial(
    pl.kernel,
    out_type=jax.ShapeDtypeStruct((NC, N_PAD, D_HALF), jnp.float32),
    mesh=_sc_mesh(),
    scratch_types=_AGG_SCRATCH,
)
def _sc_aggregate(hp_hbm, comb_hbm, out_hbm, *sc):
    ib = sc[0:NIB]
    rows = sc[NIB:NIB + 2]
    acc = sc[NIB + 2]
    isem = sc[NIB + 3:2 * NIB + 3]
    gsem = sc[2 * NIB + 3:2 * NIB + 5]
    ssem = sc[2 * NIB + 5:2 * NIB + 7]

    c = lax.axis_index("c")
    s = lax.axis_index("s")

    _fill_zero_2d(rows[0], KA, D_HALF)
    for r in range(ZROWS // KA):
        pltpu.sync_copy(rows[0], acc.at[pl.ds(s * ZROWS + r * KA, KA)])
    plsc.subcore_barrier()

    tbase = (c * NS + s) * CHUNKS

    def fire_idx(j, q):
        pltpu.async_copy(comb_hbm.at[tbase + j], ib[q], isem[q])

    def wait_idx(q):
        pltpu.make_async_copy(comb_hbm.at[tbase], ib[q], isem[q]).wait()

    def fire_gather(b, q):
        pltpu.async_copy(hp_hbm.at[ib[q].at[0]], rows[b], gsem[b])

    def wait_gather(b, q):
        pltpu.make_async_copy(hp_hbm.at[ib[q].at[0]], rows[b], gsem[b]).wait()

    def fire_scatter(b, q):
        pltpu.async_copy(rows[b], acc.at[ib[q].at[1]], ssem[b], add=True)

    def wait_scatter(b, q):
        pltpu.make_async_copy(rows[b], acc.at[ib[q].at[1]], ssem[b]).wait()

    def step(j, u, do_idx=True, do_gather=True, do_wait_prev=True):
        b, q = u % 2, u % 4
        bn, qn, qf = (u + 1) % 2, (u + 1) % 4, (u + 2) % 4
        wait_gather(b, q)
        fire_scatter(b, q)
        if do_idx:
            fire_idx(j + 2, qf)
        if do_wait_prev:
            wait_scatter(bn, qn)
        if do_gather:
            wait_idx(qn)
            fire_gather(bn, qn)

    fire_idx(0, 0)
    fire_idx(1, 1)
    wait_idx(0)
    fire_gather(0, 0)

    step(0, 0, do_wait_prev=False)
    for u in range(1, 4):
        step(u, u)

    @pl.loop(4, CHUNKS - 4, step=4)
    def _(j0):
        for u in range(4):
            step(j0 + u, u)

    step(CHUNKS - 4, 0)
    step(CHUNKS - 3, 1)
    step(CHUNKS - 2, 2, do_idx=False)
    step(CHUNKS - 1, 3, do_idx=False, do_gather=False)
    wait_scatter(1, 3)

    plsc.subcore_barrier()
    pltpu.sync_copy(acc.at[pl.ds(s * ZROWS, ZROWS)],
                    out_hbm.at[c, pl.ds(s * ZROWS, ZROWS)])


def _dinv_from(degp_ref):
    deg = degp_ref[:, 0:1] + degp_ref[:, 1:2] + 1.0
    return lax.rsqrt(deg)


def _tc1_body(x_ref, w1_ref, degp_ref, out_ref):
    dinv = _dinv_from(degp_ref)
    h = jnp.dot(x_ref[...], w1_ref[...], preferred_element_type=jnp.float32)
    hp = h * dinv
    out_ref[0] = hp[:, :D_HALF]
    out_ref[1] = hp[:, D_HALF:]


def _bn_relu(a, g_ref, be_ref):
    mean = jnp.mean(a, axis=0, keepdims=True)
    var = jnp.mean((a - mean) * (a - mean), axis=0, keepdims=True)
    zn = (a - mean) * lax.rsqrt(var + BN_EPS)
    return jnp.maximum(zn * g_ref[...][None, :] + be_ref[...][None, :], 0.0)


def _pre_bn(agg_ref, hp_ref, dinv, b_ref):
    a_lo = (agg_ref[0, :N_NODES, :] + hp_ref[0]) * dinv
    a_hi = (agg_ref[1, :N_NODES, :] + hp_ref[1]) * dinv
    return jnp.concatenate([a_lo, a_hi], axis=1) + b_ref[...][None, :]


def _tc2_body(agg_ref, hp_ref, degp_ref, b1_ref, g1_ref, be1_ref, w2_ref,
              out_ref):
    dinv = _dinv_from(degp_ref)
    a = _pre_bn(agg_ref, hp_ref, dinv, b1_ref)
    z = _bn_relu(a, g1_ref, be1_ref)
    h2 = jnp.dot(z, w2_ref[...], preferred_element_type=jnp.float32)
    hp2 = h2 * dinv
    out_ref[0] = hp2[:, :D_HALF]
    out_ref[1] = hp2[:, D_HALF:]


def _tc3_body(agg_ref, hp_ref, degp_ref, b2_ref, g2_ref, be2_ref, out_ref):
    dinv = _dinv_from(degp_ref)
    a = _pre_bn(agg_ref, hp_ref, dinv, b2_ref)
    out_ref[...] = _bn_relu(a, g2_ref, be2_ref)


def _tc_call(body, n_in, out_shape):
    return pl.pallas_call(
        body,
        out_shape=out_shape,
        in_specs=[pl.BlockSpec(memory_space=pltpu.VMEM)] * n_in,
        out_specs=pl.BlockSpec(memory_space=pltpu.VMEM)
        if not isinstance(out_shape, (list, tuple)) else
        [pl.BlockSpec(memory_space=pltpu.VMEM)] * len(out_shape),
    )


def kernel(x, edge_index, W1, b1, g1, be1, W2, b2, g2, be2):
    src = edge_index[0].astype(jnp.int32)
    dst = edge_index[1].astype(jnp.int32)

    pad = E_TILE_PAD - E_PER_TILE_AGG
    srcp = jnp.pad(src.reshape(NS, E_PER_TILE_AGG), ((0, 0), (0, pad)),
                   constant_values=0).reshape(NS, CHUNKS, KA)
    dstp = jnp.pad(dst.reshape(NS, E_PER_TILE_AGG), ((0, 0), (0, pad)),
                   constant_values=N_NODES).reshape(NS, CHUNKS, KA)
    comb = jnp.stack([
        jnp.stack([srcp, dstp], axis=2),
        jnp.stack([srcp + N_NODES, dstp], axis=2),
    ]).reshape(NC * NS * CHUNKS, 2, KA)

    degp = _sc_degree(dst).reshape(NC, N_PAD)
    degp2 = degp[:, :N_NODES].T

    hp1 = _tc_call(_tc1_body, 3,
                   jax.ShapeDtypeStruct((NC, N_NODES, D_HALF), jnp.float32))(
                       x, W1, degp2)
    agg1 = _sc_aggregate(hp1.reshape(NC * N_NODES, D_HALF), comb)

    hp2 = _tc_call(_tc2_body, 7,
                   jax.ShapeDtypeStruct((NC, N_NODES, D_HALF), jnp.float32))(
                       agg1, hp1, degp2, b1, g1, be1, W2)
    agg2 = _sc_aggregate(hp2.reshape(NC * N_NODES, D_HALF), comb)

    out = _tc_call(_tc3_body, 6,
                   jax.ShapeDtypeStruct((N_NODES, D_HID), jnp.float32))(
                       agg2, hp2, degp2, b2, g2, be2)
    return out


# speedup vs baseline: 1.0324x; 1.0008x over previous
import functools

import jax
import jax.numpy as jnp
from jax import lax
from jax.experimental import pallas as pl
from jax.experimental.pallas import tpu as pltpu
from jax.experimental.pallas import tpu_sc as plsc

N_NODES = 10000
N_PAD = 10240
N_EDGES = 320000
D_IN = 128
D_HID = 256
D_HALF = 128
BN_EPS = 1e-5

NC = 2
NS = 16
K = 80
KA = 128
CHUNKS = 160
E_TILE_PAD = CHUNKS * KA

E_PER_TILE_DEG = N_EDGES // (NC * NS)
E_PER_TILE_AGG = N_EDGES // NS
ZROWS = N_PAD // NS


def _fill_zero_2d(buf, rows, cols):
    z = jnp.zeros((16,), jnp.float32)

    @pl.loop(0, rows)
    def _(i):
        for j in range(cols // 16):
            buf[i, pl.ds(16 * j, 16)] = z


def _sc_mesh():
    return plsc.VectorSubcoreMesh(core_axis_name="c", subcore_axis_name="s")


_DEG_CHUNKS = E_PER_TILE_DEG // K


@functools.partial(
    pl.kernel,
    out_type=jax.ShapeDtypeStruct((NC * N_PAD,), jnp.float32),
    mesh=_sc_mesh(),
    scratch_types=(
        [pltpu.VMEM((K,), jnp.int32) for _ in range(4)]
        + [pltpu.VMEM((K,), jnp.float32),
           pltpu.VMEM((ZROWS,), jnp.float32)]
        + [pltpu.VMEM_SHARED((N_PAD,), jnp.float32)]
        + [pltpu.SemaphoreType.DMA for _ in range(4)]
    ),
)
def _sc_degree(dst_hbm, out_hbm, *sc):
    dstv = sc[0:4]
    onesv, zbuf, acc = sc[4], sc[5], sc[6]
    isem = sc[7:11]

    c = lax.axis_index("c")
    s = lax.axis_index("s")

    one = jnp.ones((16,), jnp.float32)
    zero = jnp.zeros((16,), jnp.float32)
    for j in range(K // 16):
        onesv[pl.ds(16 * j, 16)] = one

    @pl.loop(0, ZROWS // 16)
    def _(i):
        zbuf[pl.ds(16 * i, 16)] = zero

    pltpu.sync_copy(zbuf, acc.at[pl.ds(s * ZROWS, ZROWS)])
    plsc.subcore_barrier()

    base0 = (c * NS + s) * E_PER_TILE_DEG

    def fire_idx(j, q):
        pltpu.async_copy(dst_hbm.at[pl.ds(base0 + j * K, K)], dstv[q],
                         isem[q])

    def step(j, q, do_idx=True):
        pltpu.make_async_copy(dst_hbm.at[pl.ds(base0, K)], dstv[q],
                              isem[q]).wait()
        pltpu.sync_copy(onesv, acc.at[dstv[q]], add=True)
        if do_idx:
            fire_idx(j + 3, q3(q))

    def q3(q):
        return (q + 3) % 4

    for q in range(3):
        fire_idx(q, q)
    step(0, 0)

    @pl.loop(1, _DEG_CHUNKS - 4, step=4)
    def _(j0):
        for u in range(4):
            step(j0 + u, (1 + u) % 4)

    for j in range(_DEG_CHUNKS - 4, _DEG_CHUNKS):
        step(j, j % 4, do_idx=(j + 3 < _DEG_CHUNKS))

    plsc.subcore_barrier()
    pltpu.sync_copy(acc.at[pl.ds(s * ZROWS, ZROWS)],
                    out_hbm.at[pl.ds(c * N_PAD + s * ZROWS, ZROWS)])


NIB = 4

_AGG_SCRATCH = (
    [pltpu.VMEM((2, KA), jnp.int32) for _ in range(NIB)]
    + [pltpu.VMEM((KA, D_HALF), jnp.float32) for _ in range(2)]
    + [pltpu.VMEM_SHARED((N_PAD, D_HALF), jnp.float32)]
    + [pltpu.SemaphoreType.DMA for _ in range(NIB + 4)]
)


@functools.partial(
    pl.kernel,
    out_type=jax.ShapeDtypeStruct((NC, N_PAD, D_HALF), jnp.float32),
    mesh=_sc_mesh(),
    scratch_types=_AGG_SCRATCH,
)
def _sc_aggregate(hp_hbm, comb_hbm, out_hbm, *sc):
    ib = sc[0:NIB]
    rows = sc[NIB:NIB + 2]
    acc = sc[NIB + 2]
    isem = sc[NIB + 3:2 * NIB + 3]
    gsem = sc[2 * NIB + 3:2 * NIB + 5]
    ssem = sc[2 * NIB + 5:2 * NIB + 7]

    c = lax.axis_index("c")
    s = lax.axis_index("s")

    _fill_zero_2d(rows[0], KA, D_HALF)
    for r in range(ZROWS // KA):
        pltpu.sync_copy(rows[0], acc.at[pl.ds(s * ZROWS + r * KA, KA)])
    plsc.subcore_barrier()

    tbase = (c * NS + s) * CHUNKS

    def fire_idx(j, q):
        pltpu.async_copy(comb_hbm.at[tbase + j], ib[q], isem[q])

    def wait_idx(q):
        pltpu.make_async_copy(comb_hbm.at[tbase], ib[q], isem[q]).wait()

    def fire_gather(b, q):
        pltpu.async_copy(hp_hbm.at[ib[q].at[0]], rows[b], gsem[b])

    def wait_gather(b, q):
        pltpu.make_async_copy(hp_hbm.at[ib[q].at[0]], rows[b], gsem[b]).wait()

    def fire_scatter(b, q):
        pltpu.async_copy(rows[b], acc.at[ib[q].at[1]], ssem[b], add=True)

    def wait_scatter(b, q):
        pltpu.make_async_copy(rows[b], acc.at[ib[q].at[1]], ssem[b]).wait()

    def step(j, u, do_idx=True, do_gather=True, do_wait_prev=True):
        b, q = u % 2, u % 4
        bn, qn, qf = (u + 1) % 2, (u + 1) % 4, (u + 2) % 4
        wait_gather(b, q)
        fire_scatter(b, q)
        if do_idx:
            fire_idx(j + 2, qf)
        if do_wait_prev:
            wait_scatter(bn, qn)
        if do_gather:
            wait_idx(qn)
            fire_gather(bn, qn)

    fire_idx(0, 0)
    fire_idx(1, 1)
    wait_idx(0)
    fire_gather(0, 0)

    step(0, 0, do_wait_prev=False)
    for u in range(1, 4):
        step(u, u)

    @pl.loop(4, CHUNKS - 4, step=4)
    def _(j0):
        for u in range(4):
            step(j0 + u, u)

    step(CHUNKS - 4, 0)
    step(CHUNKS - 3, 1)
    step(CHUNKS - 2, 2, do_idx=False)
    step(CHUNKS - 1, 3, do_idx=False, do_gather=False)
    wait_scatter(1, 3)

    plsc.subcore_barrier()
    pltpu.sync_copy(acc.at[pl.ds(s * ZROWS, ZROWS)],
                    out_hbm.at[c, pl.ds(s * ZROWS, ZROWS)])


def _dinv_from(degp_ref):
    deg = degp_ref[:, 0:1] + degp_ref[:, 1:2] + 1.0
    return lax.rsqrt(deg)


def _tc1_body(x_ref, w1_ref, degp_ref, out_ref):
    dinv = _dinv_from(degp_ref)
    h = jnp.dot(x_ref[...], w1_ref[...], preferred_element_type=jnp.float32)
    hp = h * dinv
    out_ref[0] = hp[:, :D_HALF]
    out_ref[1] = hp[:, D_HALF:]


def _bn_relu(a, g_ref, be_ref):
    mean = jnp.mean(a, axis=0, keepdims=True)
    var = jnp.mean((a - mean) * (a - mean), axis=0, keepdims=True)
    zn = (a - mean) * lax.rsqrt(var + BN_EPS)
    return jnp.maximum(zn * g_ref[...][None, :] + be_ref[...][None, :], 0.0)


def _pre_bn(agg_ref, hp_ref, dinv, b_ref):
    a_lo = (agg_ref[0, :N_NODES, :] + hp_ref[0]) * dinv
    a_hi = (agg_ref[1, :N_NODES, :] + hp_ref[1]) * dinv
    return jnp.concatenate([a_lo, a_hi], axis=1) + b_ref[...][None, :]


def _tc2_body(agg_ref, hp_ref, degp_ref, b1_ref, g1_ref, be1_ref, w2_ref,
              out_ref):
    dinv = _dinv_from(degp_ref)
    a = _pre_bn(agg_ref, hp_ref, dinv, b1_ref)
    z = _bn_relu(a, g1_ref, be1_ref)
    h2 = jnp.dot(z, w2_ref[...], preferred_element_type=jnp.float32)
    hp2 = h2 * dinv
    out_ref[0] = hp2[:, :D_HALF]
    out_ref[1] = hp2[:, D_HALF:]


def _tc3_body(agg_ref, hp_ref, degp_ref, b2_ref, g2_ref, be2_ref, out_ref):
    dinv = _dinv_from(degp_ref)
    a = _pre_bn(agg_ref, hp_ref, dinv, b2_ref)
    out_ref[...] = _bn_relu(a, g2_ref, be2_ref)


def _tc_call(body, n_in, out_shape):
    return pl.pallas_call(
        body,
        out_shape=out_shape,
        in_specs=[pl.BlockSpec(memory_space=pltpu.VMEM)] * n_in,
        out_specs=pl.BlockSpec(memory_space=pltpu.VMEM)
        if not isinstance(out_shape, (list, tuple)) else
        [pl.BlockSpec(memory_space=pltpu.VMEM)] * len(out_shape),
    )


def kernel(x, edge_index, W1, b1, g1, be1, W2, b2, g2, be2):
    src = edge_index[0].astype(jnp.int32)
    dst = edge_index[1].astype(jnp.int32)

    pad = E_TILE_PAD - E_PER_TILE_AGG
    srcp = jnp.pad(src.reshape(NS, E_PER_TILE_AGG), ((0, 0), (0, pad)),
                   constant_values=0).reshape(NS, CHUNKS, KA)
    dstp = jnp.pad(dst.reshape(NS, E_PER_TILE_AGG), ((0, 0), (0, pad)),
                   constant_values=N_NODES).reshape(NS, CHUNKS, KA)
    comb = jnp.stack([
        jnp.stack([srcp, dstp], axis=2),
        jnp.stack([srcp + N_NODES, dstp], axis=2),
    ]).reshape(NC * NS * CHUNKS, 2, KA)

    degp = _sc_degree(dst).reshape(NC, N_PAD)
    degp2 = degp[:, :N_NODES].T

    hp1 = _tc_call(_tc1_body, 3,
                   jax.ShapeDtypeStruct((NC, N_NODES, D_HALF), jnp.float32))(
                       x, W1, degp2)
    agg1 = _sc_aggregate(hp1.reshape(NC * N_NODES, D_HALF), comb)

    hp2 = _tc_call(_tc2_body, 7,
                   jax.ShapeDtypeStruct((NC, N_NODES, D_HALF), jnp.float32))(
                       agg1, hp1, degp2, b1, g1, be1, W2)
    agg2 = _sc_aggregate(hp2.reshape(NC * N_NODES, D_HALF), comb)

    out = _tc_call(_tc3_body, 6,
                   jax.ShapeDtypeStruct((N_NODES, D_HID), jnp.float32))(
                       agg2, hp2, degp2, b2, g2, be2)
    return out
